# Initial kernel scaffold; baseline (speedup 1.0000x reference)
import jax
import jax.numpy as jnp
from jax.experimental import pallas as pl


def kernel(voxel_features, cu_seqlens, W1, b1, W2, b2):
    raise NotImplementedError("write your pallas kernel here")


# speedup vs baseline: 5.7286x; 5.7286x over previous
import functools

import jax
import jax.numpy as jnp
from jax import lax
from jax.experimental import pallas as pl
from jax.experimental.pallas import tpu as pltpu
from jax.experimental.pallas import tpu_sc as plsc

N = 16
M = 4096
T = 32768
F = 128
H = 256

BT = 2048
HALF = M // 2
NW = 32
L = 16
BUF = HALF + 8
T_PAD = T + HALF + BUF


def _mlp_bits_kernel(x_ref, w1_ref, b1_ref, w2_ref, b2_ref, o_ref):
    x = x_ref[...]
    h = jnp.dot(x, w1_ref[...], preferred_element_type=jnp.float32)
    h = jnp.maximum(h + b1_ref[...], 0.0)
    z = jnp.dot(h, w2_ref[...], preferred_element_type=jnp.float32)
    z = z + b2_ref[...]
    p = jax.nn.sigmoid(z)
    p = jnp.clip(p, 1e-6, 1.0 - 1e-6)
    q = jnp.where(x >= 0.0, p, 1.0 - p)
    o_ref[...] = jnp.sum(-jnp.log2(q), axis=1, keepdims=True)


def _token_bits(voxel_features, W1, b1, W2, b2):
    grid = (T // BT,)
    out = pl.pallas_call(
        _mlp_bits_kernel,
        grid=grid,
        in_specs=[
            pl.BlockSpec((BT, F), lambda i: (i, 0)),
            pl.BlockSpec((F, H), lambda i: (0, 0)),
            pl.BlockSpec((1, H), lambda i: (0, 0)),
            pl.BlockSpec((H, F), lambda i: (0, 0)),
            pl.BlockSpec((1, F), lambda i: (0, 0)),
        ],
        out_specs=pl.BlockSpec((BT, 1), lambda i: (i, 0)),
        out_shape=jax.ShapeDtypeStruct((T, 1), jnp.float32),
    )(voxel_features, W1, b1[None, :], W2, b2[None, :])
    return out[:, 0]


def _pack_kernel(bits_hbm, cu_lo_hbm, cu_hi_hbm, out_hbm,
                 cu_lo_v, cu_hi_v, buf_v, out_v):
    wid = lax.axis_index("s") * 2 + lax.axis_index("c")
    n = wid // 2
    h = wid % 2

    pltpu.sync_copy(cu_lo_hbm, cu_lo_v)
    pltpu.sync_copy(cu_hi_hbm, cu_hi_v)
    lanes = lax.iota(jnp.int32, L)
    seg_start = cu_lo_v[pl.ds(n, L)][0]
    seg_end = cu_hi_v[pl.ds(n, L)][0]
    cnt = seg_end - seg_start

    start = seg_start + h * HALF
    shift = lax.rem(start, 8)
    aligned = pl.multiple_of(start - shift, 8)
    pltpu.sync_copy(bits_hbm.at[pl.ds(aligned, BUF)], buf_v)

    m_base = h * HALF

    def body(i, _):
        vals = buf_v[pl.ds(shift + i * L, L)]
        m = m_base + i * L + lanes
        vals = jnp.where(m < cnt, vals, 0.0)
        out_v[pl.ds(i * L, L)] = vals
        return _

    lax.fori_loop(0, HALF // L, body, 0, unroll=4)
    pltpu.sync_copy(out_v, out_hbm.at[wid])


@functools.partial(jax.jit, static_argnames=())
def _pack(bits_pad, cu_lo, cu_hi):
    mesh = plsc.VectorSubcoreMesh(core_axis_name="c", subcore_axis_name="s",
                                  num_cores=2, num_subcores=16)
    return pl.kernel(
        _pack_kernel,
        out_type=jax.ShapeDtypeStruct((NW, HALF), jnp.float32),
        mesh=mesh,
        scratch_types=[
            pltpu.VMEM((2 * L,), jnp.int32),
            pltpu.VMEM((2 * L,), jnp.int32),
            pltpu.VMEM((BUF,), jnp.float32),
            pltpu.VMEM((HALF,), jnp.float32),
        ],
    )(bits_pad, cu_lo, cu_hi)


def kernel(voxel_features, cu_seqlens, W1, b1, W2, b2):
    bits = _token_bits(voxel_features, W1, b1, W2, b2)
    bits_pad = jnp.pad(bits, (0, T_PAD - T))
    cu_lo = jnp.pad(cu_seqlens[:N], (0, 2 * L - N))
    cu_hi = jnp.pad(cu_seqlens[1:N + 1], (0, 2 * L - N))
    packed = _pack(bits_pad, cu_lo, cu_hi)
    return packed.reshape(N, M)
